# baseline (speedup 1.0000x reference)
import functools

import jax
import jax.numpy as jnp
from jax import lax
from jax.experimental import pallas as pl
from jax.experimental.pallas import tpu as pltpu
from jax.experimental.pallas import tpu_sc as plsc

N_NODES = 10000
D = 128
AUG = 144
N_TILES = 32
CHUNK = 64
N_GROUP = CHUNK // 16
SB = 8
NSB = 21
N_CHUNK = SB * NSB
PER_TILE = N_CHUNK * CHUNK
E_PAD = N_TILES * PER_TILE
N_PAD = 10240
ROWS_PER_TILE = N_PAD // 16


def _front_body(x_ref, wt_ref, b_ref, aug_ref):
    h = jnp.dot(x_ref[...], wt_ref[...],
                preferred_element_type=jnp.float32,
                precision=jax.lax.Precision.HIGHEST)
    h = jnp.maximum(h + b_ref[...], 0.0)
    nrm = jnp.sqrt(jnp.sum(h * h, axis=-1, keepdims=True))
    invn = 1.0 / jnp.maximum(nrm, 1e-12)
    aug_ref[:, :D] = h
    cols = lax.broadcasted_iota(jnp.int32, (h.shape[0], AUG - D), 1)
    aug_ref[:, D:] = jnp.where(cols <= 1, invn, 0.0)


def _front(x, W1t, b1):
    n = x.shape[0]
    blk = 1000
    return pl.pallas_call(
        _front_body,
        grid=(n // blk,),
        in_specs=[pl.BlockSpec((blk, D), lambda i: (i, 0)),
                  pl.BlockSpec((D, D), lambda i: (0, 0)),
                  pl.BlockSpec((1, D), lambda i: (0, 0))],
        out_specs=pl.BlockSpec((blk, AUG), lambda i: (i, 0)),
        out_shape=jax.ShapeDtypeStruct((n, AUG), jnp.float32),
    )(x, W1t, b1[None, :])


def _mid_body(acc_ref, beta_ref, z_ref, aug_ref):
    s = acc_ref[0] + acc_ref[1]
    denom = s[:, D:D + 1]
    h = s[:, :D] / (denom + 1e-16)
    z_ref[...] = h
    nrm = jnp.sqrt(jnp.sum(h * h, axis=-1, keepdims=True))
    invn = 1.0 / jnp.maximum(nrm, 1e-12)
    aug_ref[:, :D] = h
    cols = lax.broadcasted_iota(jnp.int32, (h.shape[0], AUG - D), 1)
    invn_b = jnp.where(cols == 1, invn * beta_ref[0, 0], invn)
    aug_ref[:, D:] = jnp.where(cols <= 1, invn_b, 0.0)


def _mid(acc, beta2):
    n = N_NODES
    blk = 1000
    return pl.pallas_call(
        _mid_body,
        grid=(n // blk,),
        in_specs=[pl.BlockSpec((2, blk, AUG), lambda i: (0, i, 0)),
                  pl.BlockSpec((1, 1), lambda i: (0, 0))],
        out_specs=[pl.BlockSpec((blk, D), lambda i: (i, 0)),
                   pl.BlockSpec((blk, AUG), lambda i: (i, 0))],
        out_shape=[jax.ShapeDtypeStruct((n, D), jnp.float32),
                   jax.ShapeDtypeStruct((n, AUG), jnp.float32)],
    )(acc, beta2)


def _final_body(acc_ref, wt_ref, b_ref, z_ref, lp_ref):
    s = acc_ref[0] + acc_ref[1]
    denom = s[:, D:D + 1]
    h = s[:, :D] / (denom + 1e-16)
    z_ref[...] = h
    logits = jnp.dot(h, wt_ref[...],
                     preferred_element_type=jnp.float32,
                     precision=jax.lax.Precision.HIGHEST) + b_ref[...]
    m = jnp.max(logits, axis=-1, keepdims=True)
    lse = jnp.log(jnp.sum(jnp.exp(logits - m), axis=-1, keepdims=True)) + m
    lp_ref[...] = logits - lse


def _final(acc, W2t, b2):
    n = N_NODES
    d_out = W2t.shape[1]
    blk = 1000
    return pl.pallas_call(
        _final_body,
        grid=(n // blk,),
        in_specs=[pl.BlockSpec((2, blk, AUG), lambda i: (0, i, 0)),
                  pl.BlockSpec((D, d_out), lambda i: (0, 0)),
                  pl.BlockSpec((1, d_out), lambda i: (0, 0))],
        out_specs=[pl.BlockSpec((blk, D), lambda i: (i, 0)),
                   pl.BlockSpec((blk, d_out), lambda i: (i, 0))],
        out_shape=[jax.ShapeDtypeStruct((n, D), jnp.float32),
                   jax.ShapeDtypeStruct((n, d_out), jnp.float32)],
    )(acc, W2t, b2[None, :])


def _sc_body(e_total, aug_hbm, edata_hbm, zeros_hbm, acc_hbm,
             acc_sh, eidx, exbuf, sb0, db0, sb1, db1,
             g0, g1, s0, s1):
    cid = lax.axis_index("c")
    sid = lax.axis_index("s")
    wid = cid * 16 + sid
    ebase = wid * PER_TILE
    row0 = sid * ROWS_PER_TILE

    pltpu.sync_copy(zeros_hbm, acc_sh.at[pl.ds(row0, ROWS_PER_TILE)])
    plsc.subcore_barrier()

    lanes = lax.iota(jnp.int32, 16)
    z16i = jnp.zeros((16,), jnp.int32)
    z16f = jnp.zeros((16,), jnp.float32)

    sbufs = (sb0, sb1)
    dbufs = (db0, db1)
    gsems = (g0, g1)
    ssems = (s0, s1)

    def issue_gather(c, k):
        pltpu.async_copy(aug_hbm.at[eidx.at[c, 0]], sbufs[k], gsems[k])
        pltpu.async_copy(aug_hbm.at[eidx.at[c, 1]], dbufs[k], gsems[k])

    def wait_gather(k):
        dummy = aug_hbm.at[pl.ds(0, CHUNK)]
        pltpu.make_async_copy(dummy, sbufs[k], gsems[k]).wait()
        pltpu.make_async_copy(dummy, dbufs[k], gsems[k]).wait()

    def issue_scatter(c, k):
        pltpu.async_copy(sbufs[k], acc_sh.at[eidx.at[c, 2]], ssems[k],
                         add=True)

    def wait_scatter(k):
        pltpu.make_async_copy(sbufs[k], acc_sh.at[pl.ds(0, CHUNK)],
                              ssems[k]).wait()

    def compute_chunk(sb, c, k):
        sbuf, dbuf = sbufs[k], dbufs[k]
        gid0 = ebase + (sb * SB + c) * CHUNK
        for g in range(N_GROUP):
            riv = lanes + (g * 16)

            def dim_body(d, carry):
                acc, civ = carry
                s = plsc.load_gather(sbuf, [riv, civ])
                t = plsc.load_gather(dbuf, [riv, civ])
                return acc + s * t, civ + 1

            dotv, _ = lax.fori_loop(0, D, dim_body, (z16f, z16i), unroll=8)
            invn_s = plsc.load_gather(sbuf, [riv, z16i + D])
            invn_db = plsc.load_gather(dbuf, [riv, z16i + (D + 1)])
            alpha = dotv * invn_s * invn_db
            gid = gid0 + g * 16 + lanes
            ex = jnp.where(gid < e_total, jnp.exp(alpha), 0.0)
            exbuf[pl.ds(g * 16, 16)] = ex

        def scale_body(j, _):
            exs = plsc.load_gather(exbuf, [z16i + j])
            for kk in range(D // 16):
                sl = pl.ds(kk * 16, 16)
                sbuf[j, sl] = sbuf[j, sl] * exs
            sbuf[j, pl.ds(D, 16)] = jnp.where(lanes == 0, exs, 0.0)
            return 0

        lax.fori_loop(0, CHUNK, scale_body, 0, unroll=4)

    def sb_body(sb, _):
        pltpu.sync_copy(edata_hbm.at[wid, sb], eidx)
        issue_gather(0, 0)

        def inner(ci2, _):
            c0 = ci2 * 2
            issue_gather(c0 + 1, 1)
            wait_gather(0)
            compute_chunk(sb, c0, 0)
            issue_scatter(c0, 0)
            wait_gather(1)
            compute_chunk(sb, c0 + 1, 1)
            issue_scatter(c0 + 1, 1)

            @pl.when(ci2 < SB // 2 - 1)
            def _():
                wait_scatter(0)
                issue_gather(c0 + 2, 0)

            wait_scatter(1)
            return 0

        lax.fori_loop(0, SB // 2, inner, 0)
        wait_scatter(0)
        return 0

    lax.fori_loop(0, NSB, sb_body, 0)

    plsc.subcore_barrier()
    pltpu.sync_copy(acc_sh.at[pl.ds(row0, ROWS_PER_TILE)],
                    acc_hbm.at[cid, pl.ds(row0, ROWS_PER_TILE)])


def _sc_layer(aug, edata, zeros_slab, e_total):
    mesh = plsc.VectorSubcoreMesh(core_axis_name="c", subcore_axis_name="s")
    kern = pl.kernel(
        functools.partial(_sc_body, e_total),
        out_type=jax.ShapeDtypeStruct((2, N_PAD, AUG), jnp.float32),
        mesh=mesh,
        compiler_params=pltpu.CompilerParams(use_tc_tiling_on_sc=False,
                                             needs_layout_passes=False),
        scratch_types=[
            pltpu.VMEM_SHARED((N_PAD, AUG), jnp.float32),
            pltpu.VMEM((SB, 3, CHUNK), jnp.int32),
            pltpu.VMEM((CHUNK,), jnp.float32),
            pltpu.VMEM((CHUNK, AUG), jnp.float32),
            pltpu.VMEM((CHUNK, AUG), jnp.float32),
            pltpu.VMEM((CHUNK, AUG), jnp.float32),
            pltpu.VMEM((CHUNK, AUG), jnp.float32),
            pltpu.SemaphoreType.DMA,
            pltpu.SemaphoreType.DMA,
            pltpu.SemaphoreType.DMA,
            pltpu.SemaphoreType.DMA,
        ],
    )
    return kern(aug, edata, zeros_slab)


def kernel(x, edge_index, W1, b1, beta2, W2, b2):
    num_nodes = x.shape[0]
    n_edges = edge_index.shape[1]
    e_total = n_edges + num_nodes
    loop = jnp.arange(num_nodes, dtype=jnp.int32)
    pad = E_PAD - e_total
    src = jnp.concatenate([edge_index[0].astype(jnp.int32), loop,
                           jnp.zeros((pad,), jnp.int32)])
    dst = jnp.concatenate([edge_index[1].astype(jnp.int32), loop,
                           jnp.zeros((pad,), jnp.int32)])
    eids = jnp.arange(E_PAD, dtype=jnp.int32)
    dst_sc = jnp.where(eids < e_total, dst,
                       N_NODES + (eids % (N_PAD - N_NODES)))
    edata = jnp.stack([src.reshape(N_TILES, NSB, SB, CHUNK),
                       dst.reshape(N_TILES, NSB, SB, CHUNK),
                       dst_sc.reshape(N_TILES, NSB, SB, CHUNK)], axis=3)
    zeros_slab = jnp.zeros((ROWS_PER_TILE, AUG), jnp.float32)

    aug0 = _front(x, W1.T, b1)
    acc1 = _sc_layer(aug0, edata, zeros_slab, e_total)
    z1, aug1 = _mid(acc1, beta2.reshape(1, 1))
    acc2 = _sc_layer(aug1, edata, zeros_slab, e_total)
    z2, logp = _final(acc2, W2.T, b2)
    return (z1, z2, logp)


# speedup vs baseline: 2.0690x; 2.0690x over previous
import functools

import jax
import jax.numpy as jnp
from jax import lax
from jax.experimental import pallas as pl
from jax.experimental.pallas import tpu as pltpu
from jax.experimental.pallas import tpu_sc as plsc

N_NODES = 10000
D = 128
AUG = 144
N_TILES = 32
CHUNK = 64
N_GROUP = CHUNK // 16
SB = 8
NSB = 21
N_CHUNK = SB * NSB
PER_TILE = N_CHUNK * CHUNK
E_PAD = N_TILES * PER_TILE
N_PAD = 10240
ROWS_PER_TILE = N_PAD // 16


def _front_body(x_ref, wt_ref, b_ref, aug_ref):
    h = jnp.dot(x_ref[...], wt_ref[...],
                preferred_element_type=jnp.float32,
                precision=jax.lax.Precision.HIGHEST)
    h = jnp.maximum(h + b_ref[...], 0.0)
    nrm = jnp.sqrt(jnp.sum(h * h, axis=-1, keepdims=True))
    invn = 1.0 / jnp.maximum(nrm, 1e-12)
    aug_ref[:, :D] = h
    cols = lax.broadcasted_iota(jnp.int32, (h.shape[0], AUG - D), 1)
    aug_ref[:, D:] = jnp.where(cols <= 1, invn, 0.0)


def _front(x, W1t, b1):
    n = x.shape[0]
    blk = 1000
    return pl.pallas_call(
        _front_body,
        grid=(n // blk,),
        in_specs=[pl.BlockSpec((blk, D), lambda i: (i, 0)),
                  pl.BlockSpec((D, D), lambda i: (0, 0)),
                  pl.BlockSpec((1, D), lambda i: (0, 0))],
        out_specs=pl.BlockSpec((blk, AUG), lambda i: (i, 0)),
        out_shape=jax.ShapeDtypeStruct((n, AUG), jnp.float32),
    )(x, W1t, b1[None, :])


def _mid_body(acc_ref, beta_ref, z_ref, aug_ref):
    s = acc_ref[0] + acc_ref[1]
    denom = s[:, D:D + 1]
    h = s[:, :D] / (denom + 1e-16)
    z_ref[...] = h
    nrm = jnp.sqrt(jnp.sum(h * h, axis=-1, keepdims=True))
    invn = 1.0 / jnp.maximum(nrm, 1e-12)
    aug_ref[:, :D] = h
    cols = lax.broadcasted_iota(jnp.int32, (h.shape[0], AUG - D), 1)
    invn_b = jnp.where(cols == 1, invn * beta_ref[0, 0], invn)
    aug_ref[:, D:] = jnp.where(cols <= 1, invn_b, 0.0)


def _mid(acc, beta2):
    n = N_NODES
    blk = 1000
    return pl.pallas_call(
        _mid_body,
        grid=(n // blk,),
        in_specs=[pl.BlockSpec((2, blk, AUG), lambda i: (0, i, 0)),
                  pl.BlockSpec((1, 1), lambda i: (0, 0))],
        out_specs=[pl.BlockSpec((blk, D), lambda i: (i, 0)),
                   pl.BlockSpec((blk, AUG), lambda i: (i, 0))],
        out_shape=[jax.ShapeDtypeStruct((n, D), jnp.float32),
                   jax.ShapeDtypeStruct((n, AUG), jnp.float32)],
    )(acc, beta2)


def _final_body(acc_ref, wt_ref, b_ref, z_ref, lp_ref):
    s = acc_ref[0] + acc_ref[1]
    denom = s[:, D:D + 1]
    h = s[:, :D] / (denom + 1e-16)
    z_ref[...] = h
    logits = jnp.dot(h, wt_ref[...],
                     preferred_element_type=jnp.float32,
                     precision=jax.lax.Precision.HIGHEST) + b_ref[...]
    m = jnp.max(logits, axis=-1, keepdims=True)
    lse = jnp.log(jnp.sum(jnp.exp(logits - m), axis=-1, keepdims=True)) + m
    lp_ref[...] = logits - lse


def _final(acc, W2t, b2):
    n = N_NODES
    d_out = W2t.shape[1]
    blk = 1000
    return pl.pallas_call(
        _final_body,
        grid=(n // blk,),
        in_specs=[pl.BlockSpec((2, blk, AUG), lambda i: (0, i, 0)),
                  pl.BlockSpec((D, d_out), lambda i: (0, 0)),
                  pl.BlockSpec((1, d_out), lambda i: (0, 0))],
        out_specs=[pl.BlockSpec((blk, D), lambda i: (i, 0)),
                   pl.BlockSpec((blk, d_out), lambda i: (i, 0))],
        out_shape=[jax.ShapeDtypeStruct((n, D), jnp.float32),
                   jax.ShapeDtypeStruct((n, d_out), jnp.float32)],
    )(acc, W2t, b2[None, :])


def _sc_body(e_total, aug_hbm, edata_hbm, zeros_hbm, acc_hbm,
             acc_sh, eidx, exbuf, sb0, db0, sb1, db1,
             g0, g1, s0, s1):
    cid = lax.axis_index("c")
    sid = lax.axis_index("s")
    wid = cid * 16 + sid
    ebase = wid * PER_TILE
    row0 = sid * ROWS_PER_TILE

    pltpu.sync_copy(zeros_hbm, acc_sh.at[pl.ds(row0, ROWS_PER_TILE)])
    plsc.subcore_barrier()

    lanes = lax.iota(jnp.int32, 16)
    z16i = jnp.zeros((16,), jnp.int32)
    z16f = jnp.zeros((16,), jnp.float32)

    sbufs = (sb0, sb1)
    dbufs = (db0, db1)
    gsems = (g0, g1)
    ssems = (s0, s1)

    def issue_gather(c, k):
        pltpu.async_copy(aug_hbm.at[eidx.at[c, 0]], sbufs[k], gsems[k])
        pltpu.async_copy(aug_hbm.at[eidx.at[c, 1]], dbufs[k], gsems[k])

    def wait_gather(k):
        dummy = aug_hbm.at[pl.ds(0, CHUNK)]
        pltpu.make_async_copy(dummy, sbufs[k], gsems[k]).wait()
        pltpu.make_async_copy(dummy, dbufs[k], gsems[k]).wait()

    def issue_scatter(c, k):
        pltpu.async_copy(sbufs[k], acc_sh.at[eidx.at[c, 2]], ssems[k],
                         add=True)

    def wait_scatter(k):
        pltpu.make_async_copy(sbufs[k], acc_sh.at[pl.ds(0, CHUNK)],
                              ssems[k]).wait()

    def compute_chunk(sb, c, k):
        sbuf, dbuf = sbufs[k], dbufs[k]
        gid0 = ebase + (sb * SB + c) * CHUNK
        for g in range(N_GROUP):
            riv = lanes + (g * 16)

            def dim_body(d, carry):
                acc, civ = carry
                s = plsc.load_gather(sbuf, [riv, civ])
                t = plsc.load_gather(dbuf, [riv, civ])
                return acc + s * t, civ + 1

            dotv, _ = lax.fori_loop(0, D, dim_body, (z16f, z16i), unroll=8)
            invn_s = plsc.load_gather(sbuf, [riv, z16i + D])
            invn_db = plsc.load_gather(dbuf, [riv, z16i + (D + 1)])
            alpha = dotv * invn_s * invn_db
            gid = gid0 + g * 16 + lanes
            ex = jnp.where(gid < e_total, jnp.exp(alpha), 0.0)
            exbuf[pl.ds(g * 16, 16)] = ex

        def scale_body(j, _):
            exs = plsc.load_gather(exbuf, [z16i + j])
            for kk in range(D // 16):
                sl = pl.ds(kk * 16, 16)
                sbuf[j, sl] = sbuf[j, sl] * exs
            sbuf[j, pl.ds(D, 16)] = jnp.where(lanes == 0, exs, 0.0)
            return 0

        lax.fori_loop(0, CHUNK, scale_body, 0, unroll=4)

    def sb_body(sb, _):
        pltpu.sync_copy(edata_hbm.at[wid, sb], eidx)
        issue_gather(0, 0)

        def inner(ci2, _):
            c0 = ci2 * 2
            issue_gather(c0 + 1, 1)
            wait_gather(0)
            compute_chunk(sb, c0, 0)
            issue_scatter(c0, 0)
            wait_gather(1)
            compute_chunk(sb, c0 + 1, 1)
            issue_scatter(c0 + 1, 1)

            @pl.when(ci2 < SB // 2 - 1)
            def _():
                wait_scatter(0)
                issue_gather(c0 + 2, 0)

            wait_scatter(1)
            return 0

        lax.fori_loop(0, SB // 2, inner, 0)
        wait_scatter(0)
        return 0

    lax.fori_loop(0, NSB, sb_body, 0)

    plsc.subcore_barrier()
    pltpu.sync_copy(acc_sh.at[pl.ds(row0, ROWS_PER_TILE)],
                    acc_hbm.at[cid, pl.ds(row0, ROWS_PER_TILE)])


def _sc_layer(aug, edata, zeros_slab, e_total):
    mesh = plsc.VectorSubcoreMesh(core_axis_name="c", subcore_axis_name="s")
    kern = pl.kernel(
        functools.partial(_sc_body, e_total),
        out_type=jax.ShapeDtypeStruct((2, N_PAD, AUG), jnp.float32),
        mesh=mesh,
        compiler_params=pltpu.CompilerParams(use_tc_tiling_on_sc=False,
                                             needs_layout_passes=False),
        scratch_types=[
            pltpu.VMEM_SHARED((N_PAD, AUG), jnp.float32),
            pltpu.VMEM((SB, 3, CHUNK), jnp.int32),
            pltpu.VMEM((CHUNK,), jnp.float32),
            pltpu.VMEM((CHUNK, AUG), jnp.float32),
            pltpu.VMEM((CHUNK, AUG), jnp.float32),
            pltpu.VMEM((CHUNK, AUG), jnp.float32),
            pltpu.VMEM((CHUNK, AUG), jnp.float32),
            pltpu.SemaphoreType.DMA,
            pltpu.SemaphoreType.DMA,
            pltpu.SemaphoreType.DMA,
            pltpu.SemaphoreType.DMA,
        ],
    )
    return kern(aug, edata, zeros_slab)


def kernel(x, edge_index, W1, b1, beta2, W2, b2):
    num_nodes = x.shape[0]
    n_edges = edge_index.shape[1]
    e_total = n_edges + num_nodes
    loop = jnp.arange(num_nodes, dtype=jnp.int32)
    pad = E_PAD - e_total
    spread = (jnp.arange(pad, dtype=jnp.int32) * 97) % num_nodes
    src = jnp.concatenate([edge_index[0].astype(jnp.int32), loop, spread])
    dst = jnp.concatenate([edge_index[1].astype(jnp.int32), loop, spread])
    eids = jnp.arange(E_PAD, dtype=jnp.int32)
    dst_sc = jnp.where(eids < e_total, dst,
                       N_NODES + (eids % (N_PAD - N_NODES)))
    edata = jnp.stack([src.reshape(N_TILES, NSB, SB, CHUNK),
                       dst.reshape(N_TILES, NSB, SB, CHUNK),
                       dst_sc.reshape(N_TILES, NSB, SB, CHUNK)], axis=3)
    zeros_slab = jnp.zeros((ROWS_PER_TILE, AUG), jnp.float32)

    aug0 = _front(x, W1.T, b1)
    acc1 = _sc_layer(aug0, edata, zeros_slab, e_total)
    z1, aug1 = _mid(acc1, beta2.reshape(1, 1))
    acc2 = _sc_layer(aug1, edata, zeros_slab, e_total)
    z2, logp = _final(acc2, W2.T, b2)
    return (z1, z2, logp)


# speedup vs baseline: 3.1567x; 1.5257x over previous
import functools

import numpy as np

import jax
import jax.numpy as jnp
from jax import lax
from jax.experimental import pallas as pl
from jax.experimental.pallas import tpu as pltpu
from jax.experimental.pallas import tpu_sc as plsc

N_NODES = 10000
D = 128
AUG = 144
AUG16 = 160
N_TILES = 32
CHUNK = 48
N_GROUP = CHUNK // 16
SB = 8
NSB = 28
N_CHUNK = SB * NSB
PER_TILE = N_CHUNK * CHUNK
E_PAD = N_TILES * PER_TILE
N_PAD = 10240
ROWS_PER_TILE = N_PAD // 16

_PERM = np.arange(128).reshape(4, 2, 16).transpose(0, 2, 1).reshape(128)


def _aug16_block(h):
    nrm = jnp.sqrt(jnp.sum(h * h, axis=-1, keepdims=True))
    n = jnp.maximum(nrm, 1e-12)
    invn = 1.0 / n
    hn16 = (h * invn).astype(jnp.bfloat16)
    nhi32 = n.astype(jnp.bfloat16).astype(jnp.float32)
    nlo32 = n - nhi32
    cols = lax.broadcasted_iota(jnp.int32, (h.shape[0], AUG16 - D), 1)
    tail32 = jnp.where(cols == 0, nhi32, jnp.where(cols == 1, nlo32, 0.0))
    return hn16, tail32.astype(jnp.bfloat16)


def _front_body(x_ref, wt_ref, b_ref, aug_ref):
    h = jnp.dot(x_ref[...], wt_ref[...],
                preferred_element_type=jnp.float32,
                precision=jax.lax.Precision.HIGHEST)
    h = jnp.maximum(h + b_ref[...], 0.0)
    hn16, tail = _aug16_block(h)
    aug_ref[:, :D] = hn16
    aug_ref[:, D:] = tail


def _front(x, W1tp, b1p):
    n = x.shape[0]
    blk = 1000
    return pl.pallas_call(
        _front_body,
        grid=(n // blk,),
        in_specs=[pl.BlockSpec((blk, D), lambda i: (i, 0)),
                  pl.BlockSpec((D, D), lambda i: (0, 0)),
                  pl.BlockSpec((1, D), lambda i: (0, 0))],
        out_specs=pl.BlockSpec((blk, AUG16), lambda i: (i, 0)),
        out_shape=jax.ShapeDtypeStruct((n, AUG16), jnp.bfloat16),
    )(x, W1tp, b1p[None, :])


def _mid_body(acc_ref, z_ref, aug_ref):
    s = acc_ref[0] + acc_ref[1]
    denom = s[:, D:D + 1]
    h = s[:, :D] / (denom + 1e-16)
    z_ref[...] = h
    hn16, tail = _aug16_block(h)
    aug_ref[:, :D] = hn16
    aug_ref[:, D:] = tail


def _mid(acc):
    n = N_NODES
    blk = 1000
    return pl.pallas_call(
        _mid_body,
        grid=(n // blk,),
        in_specs=[pl.BlockSpec((2, blk, AUG), lambda i: (0, i, 0))],
        out_specs=[pl.BlockSpec((blk, D), lambda i: (i, 0)),
                   pl.BlockSpec((blk, AUG16), lambda i: (i, 0))],
        out_shape=[jax.ShapeDtypeStruct((n, D), jnp.float32),
                   jax.ShapeDtypeStruct((n, AUG16), jnp.bfloat16)],
    )(acc)


def _final_body(acc_ref, wt_ref, b_ref, z_ref, lp_ref):
    s = acc_ref[0] + acc_ref[1]
    denom = s[:, D:D + 1]
    h = s[:, :D] / (denom + 1e-16)
    z_ref[...] = h
    logits = jnp.dot(h, wt_ref[...],
                     preferred_element_type=jnp.float32,
                     precision=jax.lax.Precision.HIGHEST) + b_ref[...]
    m = jnp.max(logits, axis=-1, keepdims=True)
    lse = jnp.log(jnp.sum(jnp.exp(logits - m), axis=-1, keepdims=True)) + m
    lp_ref[...] = logits - lse


def _final(acc, W2t, b2):
    n = N_NODES
    d_out = W2t.shape[1]
    blk = 1000
    return pl.pallas_call(
        _final_body,
        grid=(n // blk,),
        in_specs=[pl.BlockSpec((2, blk, AUG), lambda i: (0, i, 0)),
                  pl.BlockSpec((D, d_out), lambda i: (0, 0)),
                  pl.BlockSpec((1, d_out), lambda i: (0, 0))],
        out_specs=[pl.BlockSpec((blk, D), lambda i: (i, 0)),
                   pl.BlockSpec((blk, d_out), lambda i: (i, 0))],
        out_shape=[jax.ShapeDtypeStruct((n, D), jnp.float32),
                   jax.ShapeDtypeStruct((n, d_out), jnp.float32)],
    )(acc, W2t, b2[None, :])


def _sc_body(e_total, aug_hbm, bvec_hbm, edata_hbm, zeros_hbm, acc_hbm,
             acc_sh, eidx, bbuf, exn_b, exd_b,
             s0b, d0b, s1b, d1b, sc0, sc1,
             g0, g1, p0, p1):
    cid = lax.axis_index("c")
    sid = lax.axis_index("s")
    wid = cid * 16 + sid
    ebase = wid * PER_TILE
    row0 = sid * ROWS_PER_TILE

    pltpu.sync_copy(bvec_hbm, bbuf)
    pltpu.sync_copy(zeros_hbm, acc_sh.at[pl.ds(row0, ROWS_PER_TILE)])
    plsc.subcore_barrier()

    lanes = lax.iota(jnp.int32, 16)
    z16i = jnp.zeros((16,), jnp.int32)
    z16f = jnp.zeros((16,), jnp.float32)

    sbufs = (s0b, s1b)
    dbufs = (d0b, d1b)
    scats = (sc0, sc1)
    gsems = (g0, g1)
    ssems = (p0, p1)

    def issue_gather(c, k):
        pltpu.async_copy(aug_hbm.at[eidx.at[c, 0]], sbufs[k], gsems[k])
        pltpu.async_copy(aug_hbm.at[eidx.at[c, 1]], dbufs[k], gsems[k])

    def wait_gather(k):
        dummy = aug_hbm.at[pl.ds(0, CHUNK)]
        pltpu.make_async_copy(dummy, sbufs[k], gsems[k]).wait()
        pltpu.make_async_copy(dummy, dbufs[k], gsems[k]).wait()

    def issue_scatter(c, k):
        pltpu.async_copy(scats[k], acc_sh.at[eidx.at[c, 2]], ssems[k],
                         add=True)

    def wait_scatter(k):
        pltpu.make_async_copy(scats[k], acc_sh.at[pl.ds(0, CHUNK)],
                              ssems[k]).wait()

    def compute_chunk(sb, c, k):
        sbuf, dbuf, scat = sbufs[k], dbufs[k], scats[k]
        gid0 = ebase + (sb * SB + c) * CHUNK
        beta = bbuf[...]
        for g in range(N_GROUP):

            def edge_body(jj, carry):
                dots, nrms = carry
                j = g * 16 + jj
                ps = z16f
                for kk in range(D // 32):
                    a = sbuf[j, pl.ds(32 * kk, 32)]
                    b = dbuf[j, pl.ds(32 * kk, 32)]
                    u0, u1 = plsc.unpack(a * b,
                                         format=plsc.PackFormat.INTERLEAVED)
                    ps = ps + u0 + u1
                tot = jnp.sum(ps)
                nb = sbuf[j, pl.ds(D, 32)]
                n0, n1 = plsc.unpack(nb, format=plsc.PackFormat.INTERLEAVED)
                nrm_s = jnp.sum(jnp.where(lanes == 0, n0 + n1, 0.0))
                return (jnp.where(lanes == jj, tot, dots),
                        jnp.where(lanes == jj, nrm_s, nrms))

            dots, nrms = lax.fori_loop(0, 16, edge_body, (z16f, z16f),
                                       unroll=2)
            gid = gid0 + g * 16 + lanes
            ex = jnp.where(gid < e_total, jnp.exp(dots * beta), 0.0)
            exd_b[pl.ds(g * 16, 16)] = ex
            exn_b[pl.ds(g * 16, 16)] = ex * nrms

        def scale_body(j, _):
            exs = plsc.load_gather(exn_b, [z16i + j])
            exd = plsc.load_gather(exd_b, [z16i + j])
            for kk in range(D // 32):
                a = sbuf[j, pl.ds(32 * kk, 32)]
                u0, u1 = plsc.unpack(a, format=plsc.PackFormat.INTERLEAVED)
                scat[j, pl.ds(32 * kk, 16)] = u0 * exs
                scat[j, pl.ds(32 * kk + 16, 16)] = u1 * exs
            scat[j, pl.ds(D, 16)] = jnp.where(lanes == 0, exd, 0.0)
            return 0

        lax.fori_loop(0, CHUNK, scale_body, 0, unroll=4)

    def sb_body(sb, _):
        pltpu.sync_copy(edata_hbm.at[wid, sb], eidx)
        issue_gather(0, 0)

        def inner(ci2, _):
            c0 = ci2 * 2
            issue_gather(c0 + 1, 1)
            wait_gather(0)
            compute_chunk(sb, c0, 0)
            issue_scatter(c0, 0)
            wait_gather(1)
            compute_chunk(sb, c0 + 1, 1)
            issue_scatter(c0 + 1, 1)

            @pl.when(ci2 < SB // 2 - 1)
            def _():
                wait_scatter(0)
                issue_gather(c0 + 2, 0)

            wait_scatter(1)
            return 0

        lax.fori_loop(0, SB // 2, inner, 0)
        wait_scatter(0)
        return 0

    lax.fori_loop(0, NSB, sb_body, 0)

    plsc.subcore_barrier()
    pltpu.sync_copy(acc_sh.at[pl.ds(row0, ROWS_PER_TILE)],
                    acc_hbm.at[cid, pl.ds(row0, ROWS_PER_TILE)])


def _sc_layer(aug16, bvec, edata, zeros_slab, e_total):
    mesh = plsc.VectorSubcoreMesh(core_axis_name="c", subcore_axis_name="s")
    kern = pl.kernel(
        functools.partial(_sc_body, e_total),
        out_type=jax.ShapeDtypeStruct((2, N_PAD, AUG), jnp.float32),
        mesh=mesh,
        compiler_params=pltpu.CompilerParams(use_tc_tiling_on_sc=False,
                                             needs_layout_passes=False),
        scratch_types=[
            pltpu.VMEM_SHARED((N_PAD, AUG), jnp.float32),
            pltpu.VMEM((SB, 3, CHUNK), jnp.int32),
            pltpu.VMEM((16,), jnp.float32),
            pltpu.VMEM((CHUNK,), jnp.float32),
            pltpu.VMEM((CHUNK,), jnp.float32),
            pltpu.VMEM((CHUNK, AUG16), jnp.bfloat16),
            pltpu.VMEM((CHUNK, AUG16), jnp.bfloat16),
            pltpu.VMEM((CHUNK, AUG16), jnp.bfloat16),
            pltpu.VMEM((CHUNK, AUG16), jnp.bfloat16),
            pltpu.VMEM((CHUNK, AUG), jnp.float32),
            pltpu.VMEM((CHUNK, AUG), jnp.float32),
            pltpu.SemaphoreType.DMA,
            pltpu.SemaphoreType.DMA,
            pltpu.SemaphoreType.DMA,
            pltpu.SemaphoreType.DMA,
        ],
    )
    return kern(aug16, bvec, edata, zeros_slab)


def kernel(x, edge_index, W1, b1, beta2, W2, b2):
    num_nodes = x.shape[0]
    n_edges = edge_index.shape[1]
    e_total = n_edges + num_nodes
    loop = jnp.arange(num_nodes, dtype=jnp.int32)
    pad = E_PAD - e_total
    spread = (jnp.arange(pad, dtype=jnp.int32) * 97) % num_nodes
    src = jnp.concatenate([edge_index[0].astype(jnp.int32), loop, spread])
    dst = jnp.concatenate([edge_index[1].astype(jnp.int32), loop, spread])
    eids = jnp.arange(E_PAD, dtype=jnp.int32)
    dst_sc = jnp.where(eids < e_total, dst,
                       N_NODES + (eids % (N_PAD - N_NODES)))
    edata = jnp.stack([src.reshape(N_TILES, NSB, SB, CHUNK),
                       dst.reshape(N_TILES, NSB, SB, CHUNK),
                       dst_sc.reshape(N_TILES, NSB, SB, CHUNK)], axis=3)
    zeros_slab = jnp.zeros((ROWS_PER_TILE, AUG), jnp.float32)
    perm = jnp.asarray(_PERM)

    one_vec = jnp.ones((16,), jnp.float32)
    beta_vec = jnp.broadcast_to(beta2[0], (16,)).astype(jnp.float32)

    aug0 = _front(x, W1.T[:, perm], b1[perm])
    acc1 = _sc_layer(aug0, one_vec, edata, zeros_slab, e_total)
    z1, aug1n = _mid(acc1)
    aug1 = jnp.concatenate([aug1n[:, perm], aug1n[:, D:]], axis=1)
    acc2 = _sc_layer(aug1, beta_vec, edata, zeros_slab, e_total)
    z2, logp = _final(acc2, W2.T, b2)
    return (z1, z2, logp)
